# baseline (speedup 1.0000x reference)
import functools

import jax
import jax.numpy as jnp
from jax import lax
from jax.experimental import pallas as pl
from jax.experimental.pallas import tpu as pltpu
from jax.experimental.pallas import tpu_sc as plsc

_CB = 1024
_D = 256
_B = 16
_HW = 1024
_N = _B * _HW
_BETA = 0.25

_NC = 2
_NS = 16
_L = 16
_NW = _NC * _NS
_CPW = _D // _NW


def _vq_body(z_ref, w_ref, idx_ref, loss_ref, perp_ref, mdist_ref,
             counts_ref, acc_ref):
    b = pl.program_id(0)
    w = w_ref[...]

    @pl.when(b == 0)
    def _init():
        counts_ref[...] = jnp.zeros_like(counts_ref)
        acc_ref[...] = jnp.zeros_like(acc_ref)

    zb = z_ref[0]
    zsq = jnp.sum(zb * zb, axis=0)
    wsq = jnp.sum(w * w, axis=1)
    mm = lax.dot_general(w, zb, (((1,), (0,)), ((), ())),
                         preferred_element_type=jnp.float32)
    d = (zsq[None, :] + wsq[:, None]) - 2.0 * mm
    dmin = jnp.min(d, axis=0)
    code_iota = lax.broadcasted_iota(jnp.int32, (_CB, _HW), 0)
    idx = jnp.min(jnp.where(d == dmin[None, :], code_iota, _CB), axis=0)
    oh = (code_iota == idx[None, :]).astype(jnp.float32)
    idx_ref[0, 0] = idx
    counts_ref[...] += jnp.sum(oh, axis=1, keepdims=True)
    acc_ref[0, :] += jnp.broadcast_to(jnp.sum(dmin), (128,))
    acc_ref[1, :] += jnp.broadcast_to(jnp.sum(d), (128,))

    @pl.when(b == _B - 1)
    def _final():
        loss_sum = acc_ref[0, 0]
        dist_sum = acc_ref[1, 0]
        loss_ref[...] = jnp.full((8, 128), (1.0 + _BETA) * loss_sum / (_N * _D))
        mdist_ref[...] = jnp.full((8, 128), dist_sum / (_N * _CB))
        e_mean = counts_ref[...] * (1.0 / _N)
        ent = -jnp.sum(e_mean * jnp.log(e_mean + 1e-10))
        perp_ref[...] = jnp.full((8, 128), jnp.exp(ent))


def _sc_gather_body(wt_hbm, idx_hbm, zq_hbm, wt_v, idx_v, out_v):
    wid = lax.axis_index("s") * _NC + lax.axis_index("c")
    c0 = wid * _CPW
    pltpu.sync_copy(wt_hbm.at[pl.ds(c0 * _CB, _CPW * _CB)], wt_v)
    pltpu.sync_copy(idx_hbm, idx_v)

    def batch_body(b, carry):
        def j_body(j, carry2):
            ids = idx_v[pl.ds(b * _HW + j * _L, _L)]
            for c in range(_CPW):
                out_v[pl.ds(c * _HW + j * _L, _L)] = plsc.load_gather(
                    wt_v, [ids + (c * _CB)])
            return carry2
        lax.fori_loop(0, _HW // _L, j_body, 0)
        pltpu.sync_copy(out_v,
                        zq_hbm.at[pl.ds(b * (_D * _HW) + c0 * _HW, _CPW * _HW)])
        return carry
    lax.fori_loop(0, _B, batch_body, 0)


def _sc_gather(wt_flat, idx_flat):
    mesh = plsc.VectorSubcoreMesh(core_axis_name="c", subcore_axis_name="s",
                                  num_cores=_NC)
    f = functools.partial(
        pl.kernel, mesh=mesh,
        compiler_params=pltpu.CompilerParams(needs_layout_passes=False),
        out_type=jax.ShapeDtypeStruct((_B * _D * _HW,), jnp.float32),
        scratch_types=[
            pltpu.VMEM((_CPW * _CB,), jnp.float32),
            pltpu.VMEM((_N,), jnp.int32),
            pltpu.VMEM((_CPW * _HW,), jnp.float32),
        ],
    )(_sc_gather_body)
    return f(wt_flat, idx_flat)


def kernel(z, weight):
    z3 = z.reshape(_B, _D, _HW)
    out_shapes = (
        jax.ShapeDtypeStruct((_B, 1, _HW), jnp.int32),
        jax.ShapeDtypeStruct((8, 128), jnp.float32),
        jax.ShapeDtypeStruct((8, 128), jnp.float32),
        jax.ShapeDtypeStruct((8, 128), jnp.float32),
    )
    idx3, loss, perp, mdist = pl.pallas_call(
        _vq_body,
        grid=(_B,),
        in_specs=[
            pl.BlockSpec((1, _D, _HW), lambda b: (b, 0, 0)),
            pl.BlockSpec((_CB, _D), lambda b: (0, 0)),
        ],
        out_specs=(
            pl.BlockSpec((1, 1, _HW), lambda b: (b, 0, 0)),
            pl.BlockSpec((8, 128), lambda b: (0, 0)),
            pl.BlockSpec((8, 128), lambda b: (0, 0)),
            pl.BlockSpec((8, 128), lambda b: (0, 0)),
        ),
        out_shape=out_shapes,
        scratch_shapes=[
            pltpu.VMEM((_CB, 1), jnp.float32),
            pltpu.VMEM((2, 128), jnp.float32),
        ],
    )(z3, weight)
    wt = jnp.transpose(weight).reshape(-1)
    zq_flat = _sc_gather(wt, idx3.reshape(_N))
    return (zq_flat.reshape(_B, _D, 32, 32), loss[0, 0], perp[0, 0],
            idx3.reshape(_N, 1), mdist[0, 0])


# speedup vs baseline: 1.0144x; 1.0144x over previous
import functools

import jax
import jax.numpy as jnp
from jax import lax
from jax.experimental import pallas as pl
from jax.experimental.pallas import tpu as pltpu
from jax.experimental.pallas import tpu_sc as plsc

_CB = 1024
_D = 256
_B = 16
_HW = 1024
_N = _B * _HW
_BETA = 0.25

_NC = 2
_NS = 16
_L = 16
_NW = _NC * _NS
_CPW = _D // _NW


def _vq_body(z_ref, w_ref, idx_ref, loss_ref, perp_ref, mdist_ref, wt_ref,
             counts_ref, acc_ref):
    b = pl.program_id(0)
    w = w_ref[...]

    @pl.when(b == 0)
    def _init():
        counts_ref[...] = jnp.zeros_like(counts_ref)
        acc_ref[...] = jnp.zeros_like(acc_ref)
        eye = (lax.broadcasted_iota(jnp.int32, (_CB, _CB), 0)
               == lax.broadcasted_iota(jnp.int32, (_CB, _CB), 1)
               ).astype(jnp.float32)
        wt_ref[...] = lax.dot_general(w, eye, (((0,), (0,)), ((), ())),
                                      preferred_element_type=jnp.float32)

    zb = z_ref[0]
    zsq = jnp.sum(zb * zb, axis=0)
    wsq = jnp.sum(w * w, axis=1)
    mm = lax.dot_general(w, zb, (((1,), (0,)), ((), ())),
                         preferred_element_type=jnp.float32)
    d = (zsq[None, :] + wsq[:, None]) - 2.0 * mm
    dmin = jnp.min(d, axis=0)
    code_iota = lax.broadcasted_iota(jnp.int32, (_CB, _HW), 0)
    idx = jnp.min(jnp.where(d == dmin[None, :], code_iota, _CB), axis=0)
    oh = (code_iota == idx[None, :]).astype(jnp.float32)
    idx_ref[0, 0] = idx
    counts_ref[...] += jnp.sum(oh, axis=1, keepdims=True)
    acc_ref[0, :] += jnp.broadcast_to(jnp.sum(dmin), (128,))
    acc_ref[1, :] += jnp.broadcast_to(jnp.sum(d), (128,))

    @pl.when(b == _B - 1)
    def _final():
        loss_sum = acc_ref[0, 0]
        dist_sum = acc_ref[1, 0]
        loss_ref[...] = jnp.full((8, 128), (1.0 + _BETA) * loss_sum / (_N * _D))
        mdist_ref[...] = jnp.full((8, 128), dist_sum / (_N * _CB))
        e_mean = counts_ref[...] * (1.0 / _N)
        ent = -jnp.sum(e_mean * jnp.log(e_mean + 1e-10))
        perp_ref[...] = jnp.full((8, 128), jnp.exp(ent))


_UNROLL = 4


def _sc_gather_body(wt_hbm, idx_hbm, zq_hbm, wt_v, idx_v, out0, out1,
                    sem0, sem1):
    wid = lax.axis_index("s") * _NC + lax.axis_index("c")
    c0 = wid * _CPW
    pltpu.sync_copy(wt_hbm.at[pl.ds(c0 * _CB, _CPW * _CB)], wt_v)
    pltpu.sync_copy(idx_hbm, idx_v)
    outs = (out0, out1)
    sems = (sem0, sem1)
    handles = [None, None]

    for b in range(_B):
        out_v = outs[b % 2]
        if handles[b % 2] is not None:
            handles[b % 2].wait()

        def j_body(j, carry, b=b, out_v=out_v):
            base = b * _HW + j * (_L * _UNROLL)
            for u in range(_UNROLL):
                ids = idx_v[pl.ds(base + u * _L, _L)]
                for c in range(_CPW):
                    out_v[pl.ds(c * _HW + j * (_L * _UNROLL) + u * _L, _L)] = (
                        plsc.load_gather(wt_v, [ids + (c * _CB)]))
            return carry
        lax.fori_loop(0, _HW // (_L * _UNROLL), j_body, 0)
        handles[b % 2] = pltpu.make_async_copy(
            out_v,
            zq_hbm.at[pl.ds(b * (_D * _HW) + c0 * _HW, _CPW * _HW)],
            sems[b % 2])
        handles[b % 2].start()
    handles[0].wait()
    handles[1].wait()


def _sc_gather(wt_flat, idx_flat):
    mesh = plsc.VectorSubcoreMesh(core_axis_name="c", subcore_axis_name="s",
                                  num_cores=_NC)
    f = functools.partial(
        pl.kernel, mesh=mesh,
        compiler_params=pltpu.CompilerParams(needs_layout_passes=False),
        out_type=jax.ShapeDtypeStruct((_B * _D * _HW,), jnp.float32),
        scratch_types=[
            pltpu.VMEM((_CPW * _CB,), jnp.float32),
            pltpu.VMEM((_N,), jnp.int32),
            pltpu.VMEM((_CPW * _HW,), jnp.float32),
            pltpu.VMEM((_CPW * _HW,), jnp.float32),
            pltpu.SemaphoreType.DMA,
            pltpu.SemaphoreType.DMA,
        ],
    )(_sc_gather_body)
    return f(wt_flat, idx_flat)


def kernel(z, weight):
    z3 = z.reshape(_B, _D, _HW)
    out_shapes = (
        jax.ShapeDtypeStruct((_B, 1, _HW), jnp.int32),
        jax.ShapeDtypeStruct((8, 128), jnp.float32),
        jax.ShapeDtypeStruct((8, 128), jnp.float32),
        jax.ShapeDtypeStruct((8, 128), jnp.float32),
        jax.ShapeDtypeStruct((_D, _CB), jnp.float32),
    )
    idx3, loss, perp, mdist, wt = pl.pallas_call(
        _vq_body,
        grid=(_B,),
        in_specs=[
            pl.BlockSpec((1, _D, _HW), lambda b: (b, 0, 0)),
            pl.BlockSpec((_CB, _D), lambda b: (0, 0)),
        ],
        out_specs=(
            pl.BlockSpec((1, 1, _HW), lambda b: (b, 0, 0)),
            pl.BlockSpec((8, 128), lambda b: (0, 0)),
            pl.BlockSpec((8, 128), lambda b: (0, 0)),
            pl.BlockSpec((8, 128), lambda b: (0, 0)),
            pl.BlockSpec((_D, _CB), lambda b: (0, 0)),
        ),
        out_shape=out_shapes,
        scratch_shapes=[
            pltpu.VMEM((_CB, 1), jnp.float32),
            pltpu.VMEM((2, 128), jnp.float32),
        ],
    )(z3, weight)
    zq_flat = _sc_gather(wt.reshape(-1), idx3.reshape(_N))
    return (zq_flat.reshape(_B, _D, 32, 32), loss[0, 0], perp[0, 0],
            idx3.reshape(_N, 1), mdist[0, 0])
